# Initial kernel scaffold; baseline (speedup 1.0000x reference)
import jax
import jax.numpy as jnp
from jax.experimental import pallas as pl


def kernel(x, table):
    raise NotImplementedError("write your pallas kernel here")


# speedup vs baseline: 1.8314x; 1.8314x over previous
import functools

import jax
import jax.numpy as jnp
from jax import lax
from jax.experimental import pallas as pl
from jax.experimental.pallas import tpu as pltpu
from jax.experimental.pallas import tpu_sc as plsc

_DIM = 64
_NW = 32
_CHUNK = 512


@functools.partial(jax.jit, static_argnums=())
def kernel(x, table):
    b, l = x.shape
    total = b * l
    dim = table.shape[1]
    per_w = total // _NW
    nchunk = per_w // _CHUNK
    flat_idx = x.reshape(total).astype(jnp.int32)

    mesh = plsc.VectorSubcoreMesh(core_axis_name="c", subcore_axis_name="s")

    @functools.partial(
        pl.kernel,
        mesh=mesh,
        out_type=jax.ShapeDtypeStruct((total, dim), jnp.float32),
        scratch_types=[
            pltpu.VMEM((per_w,), jnp.int32),
            pltpu.VMEM((_CHUNK, dim), jnp.float32),
            pltpu.SemaphoreType.DMA,
        ],
        compiler_params=pltpu.CompilerParams(use_tc_tiling_on_sc=False),
    )
    def gather_kernel(idx_hbm, table_hbm, out_hbm, idx_v, rows_v, sem):
        wid = lax.axis_index("s") * 2 + lax.axis_index("c")
        base = wid * per_w
        pltpu.sync_copy(idx_hbm.at[pl.ds(base, per_w)], idx_v)

        def body(i, carry):
            off = i * _CHUNK
            pltpu.async_copy(
                table_hbm.at[idx_v.at[pl.ds(off, _CHUNK)]], rows_v, sem
            ).wait()
            pltpu.sync_copy(rows_v, out_hbm.at[pl.ds(base + off, _CHUNK)])
            return carry

        lax.fori_loop(0, nchunk, body, 0)

    out = gather_kernel(flat_idx, table)
    return out.reshape(b, l, dim)


# speedup vs baseline: 1.8676x; 1.0198x over previous
import functools

import jax
import jax.numpy as jnp
from jax import lax
from jax.experimental import pallas as pl
from jax.experimental.pallas import tpu as pltpu
from jax.experimental.pallas import tpu_sc as plsc

_NW = 32
_CHUNK = 256
_NBUF = 4


@jax.jit
def kernel(x, table):
    b, l = x.shape
    total = b * l
    dim = table.shape[1]
    per_w = total // _NW
    nchunk = per_w // _CHUNK
    ngrp = nchunk // _NBUF
    flat_idx = x.reshape(total).astype(jnp.int32)

    mesh = plsc.VectorSubcoreMesh(core_axis_name="c", subcore_axis_name="s")

    @functools.partial(
        pl.kernel,
        mesh=mesh,
        out_type=jax.ShapeDtypeStruct((total, dim), jnp.float32),
        scratch_types=[
            pltpu.VMEM((per_w,), jnp.int32),
            pltpu.VMEM((_NBUF, _CHUNK, dim), jnp.float32),
            [pltpu.SemaphoreType.DMA] * _NBUF,
            [pltpu.SemaphoreType.DMA] * _NBUF,
        ],
        compiler_params=pltpu.CompilerParams(use_tc_tiling_on_sc=False),
    )
    def gather_kernel(idx_hbm, table_hbm, out_hbm, idx_v, rows_v, gsems, ssems):
        wid = lax.axis_index("s") * 2 + lax.axis_index("c")
        base = wid * per_w
        pltpu.sync_copy(idx_hbm.at[pl.ds(base, per_w)], idx_v)

        def gather_desc(c, buf):
            return pltpu.make_async_copy(
                table_hbm.at[idx_v.at[pl.ds(c * _CHUNK, _CHUNK)]],
                rows_v.at[buf],
                gsems[buf],
            )

        def scatter_desc(c, buf):
            return pltpu.make_async_copy(
                rows_v.at[buf],
                out_hbm.at[pl.ds(base + c * _CHUNK, _CHUNK)],
                ssems[buf],
            )

        for buf in range(_NBUF):
            gather_desc(buf, buf).start()

        def body(j, carry):
            c0 = j * _NBUF
            for buf in range(_NBUF):
                gather_desc(c0 + buf, buf).wait()
                scatter_desc(c0 + buf, buf).start()
            for buf in range(_NBUF):
                @pl.when(j < ngrp - 1)
                def _(buf=buf):
                    scatter_desc(c0 + buf, buf).wait()
                    gather_desc(c0 + _NBUF + buf, buf).start()
            return carry

        lax.fori_loop(0, ngrp, body, 0)

        for buf in range(_NBUF):
            scatter_desc((ngrp - 1) * _NBUF + buf, buf).wait()

    out = gather_kernel(flat_idx, table)
    return out.reshape(b, l, dim)


# speedup vs baseline: 1.8735x; 1.0032x over previous
import functools

import jax
import jax.numpy as jnp
from jax import lax
from jax.experimental import pallas as pl
from jax.experimental.pallas import tpu as pltpu
from jax.experimental.pallas import tpu_sc as plsc

_NW = 32
_CHUNK = 128
_NBUF = 8


@jax.jit
def kernel(x, table):
    b, l = x.shape
    total = b * l
    dim = table.shape[1]
    per_w = total // _NW
    nchunk = per_w // _CHUNK
    ngrp = nchunk // _NBUF
    flat_idx = x.reshape(total).astype(jnp.int32)

    mesh = plsc.VectorSubcoreMesh(core_axis_name="c", subcore_axis_name="s")

    @functools.partial(
        pl.kernel,
        mesh=mesh,
        out_type=jax.ShapeDtypeStruct((total, dim), jnp.float32),
        scratch_types=[
            pltpu.VMEM((per_w,), jnp.int32),
            pltpu.VMEM((_NBUF, _CHUNK, dim), jnp.float32),
            [pltpu.SemaphoreType.DMA] * _NBUF,
            [pltpu.SemaphoreType.DMA] * _NBUF,
        ],
        compiler_params=pltpu.CompilerParams(use_tc_tiling_on_sc=False),
    )
    def gather_kernel(idx_hbm, table_hbm, out_hbm, idx_v, rows_v, gsems, ssems):
        wid = lax.axis_index("s") * 2 + lax.axis_index("c")
        base = wid * per_w
        pltpu.sync_copy(idx_hbm.at[pl.ds(base, per_w)], idx_v)

        def gather_desc(c, buf):
            return pltpu.make_async_copy(
                table_hbm.at[idx_v.at[pl.ds(c * _CHUNK, _CHUNK)]],
                rows_v.at[buf],
                gsems[buf],
            )

        def scatter_desc(c, buf):
            return pltpu.make_async_copy(
                rows_v.at[buf],
                out_hbm.at[pl.ds(base + c * _CHUNK, _CHUNK)],
                ssems[buf],
            )

        for buf in range(_NBUF):
            gather_desc(buf, buf).start()

        def body(j, carry):
            c0 = j * _NBUF
            for buf in range(_NBUF):
                gather_desc(c0 + buf, buf).wait()
                scatter_desc(c0 + buf, buf).start()
            for buf in range(_NBUF):
                @pl.when(j < ngrp - 1)
                def _(buf=buf):
                    scatter_desc(c0 + buf, buf).wait()
                    gather_desc(c0 + _NBUF + buf, buf).start()
            return carry

        lax.fori_loop(0, ngrp, body, 0)

        for buf in range(_NBUF):
            scatter_desc((ngrp - 1) * _NBUF + buf, buf).wait()

    out = gather_kernel(flat_idx, table)
    return out.reshape(b, l, dim)
